# baseline (speedup 1.0000x reference)
import functools

import jax
import jax.numpy as jnp
import numpy as np
from jax import lax
from jax.experimental import pallas as pl
from jax.experimental.pallas import tpu as pltpu
from jax.experimental.pallas import tpu_sc as plsc

_B, _S, _D = 4096, 200, 64
_ROW = _S * _D
_W = 512
_SUB_PER_ROW = _ROW // _W
_NSUB = _B * _SUB_PER_ROW
_NC, _NS = 2, 16
_NW = _NC * _NS
_SUB_PER_WORKER = _NSUB // _NW
_G = 8
_NGROUPS = _SUB_PER_WORKER // _G


def _build_tables():
    key = jax.random.key(42)
    k_mask, k_perm = jax.random.split(key)
    mask = np.asarray(jax.random.bernoulli(k_mask, 0.1, (_B, _S, _D)))
    perm = np.asarray(jax.random.permutation(k_perm, _B)).astype(np.int32)
    m = mask.reshape(_NSUB, 32, 16).astype(np.uint32)
    packed = (m << np.arange(32, dtype=np.uint32)[None, :, None]).sum(
        axis=1, dtype=np.uint32)
    pidx = (perm[:, None] * _SUB_PER_ROW
            + np.arange(_SUB_PER_ROW, dtype=np.int32)[None, :])
    return packed, pidx.reshape(_NSUB).astype(np.int32)


_PMASK, _PIDX = _build_tables()


@functools.cache
def _make_swap_kernel():
    mesh = plsc.VectorSubcoreMesh(core_axis_name="c", subcore_axis_name="s")

    @functools.partial(
        pl.kernel,
        mesh=mesh,
        out_type=jax.ShapeDtypeStruct((_NSUB, _W), jnp.float32),
        scratch_types=[
            pltpu.VMEM((_SUB_PER_WORKER,), jnp.int32),
            pltpu.VMEM((_G, _W), jnp.float32),
            pltpu.VMEM((_G, _W), jnp.float32),
            pltpu.VMEM((_G, _W), jnp.float32),
            pltpu.VMEM((_G, 16), jnp.uint32),
            pltpu.SemaphoreType.DMA,
        ],
    )
    def _swap_kernel(x_hbm, pm_hbm, pidx_hbm, out_hbm,
                     idx_v, x_v, swap_v, out_v, p_v, sem):
        wid = lax.axis_index("s") * _NC + lax.axis_index("c")
        base = wid * _SUB_PER_WORKER
        pltpu.sync_copy(pidx_hbm.at[pl.ds(base, _SUB_PER_WORKER)], idx_v)

        def group(g, carry):
            goff = pl.multiple_of(g * _G, _G)
            row0 = pl.multiple_of(base + g * _G, _G)
            gather = pltpu.async_copy(x_hbm.at[idx_v.at[pl.ds(goff, _G)]],
                                      swap_v, sem)
            pltpu.sync_copy(x_hbm.at[pl.ds(row0, _G)], x_v)
            pltpu.sync_copy(pm_hbm.at[pl.ds(row0, _G)], p_v)
            gather.wait()
            for r in range(_G):
                pvec = p_v[r, :]
                for c in range(32):
                    off = c * 16
                    xc = x_v[r, pl.ds(off, 16)]
                    sc = swap_v[r, pl.ds(off, 16)]
                    m = (pvec & jnp.uint32(1 << c)) != 0
                    out_v[r, pl.ds(off, 16)] = jnp.where(
                        m & (xc != 0.0), sc, xc)
            pltpu.sync_copy(out_v, out_hbm.at[pl.ds(row0, _G)])
            return carry

        lax.fori_loop(0, _NGROUPS, group, 0)

    return _swap_kernel


def kernel(inputs):
    x = inputs.reshape(_NSUB, _W)
    out = _make_swap_kernel()(x, jnp.asarray(_PMASK), jnp.asarray(_PIDX))
    return out.reshape(_B, _S, _D)


# speedup vs baseline: 1.3961x; 1.3961x over previous
import functools

import jax
import jax.numpy as jnp
import numpy as np
from jax import lax
from jax.experimental import pallas as pl
from jax.experimental.pallas import tpu as pltpu
from jax.experimental.pallas import tpu_sc as plsc

_B, _S, _D = 4096, 200, 64
_ROW = _S * _D
_W = 512
_SUB_PER_ROW = _ROW // _W
_NSUB = _B * _SUB_PER_ROW
_NC, _NS = 2, 16
_NW = _NC * _NS
_SUB_PER_WORKER = _NSUB // _NW
_G = 8
_NGROUPS = _SUB_PER_WORKER // _G


def _build_tables():
    key = jax.random.key(42)
    k_mask, k_perm = jax.random.split(key)
    mask = np.asarray(jax.random.bernoulli(k_mask, 0.1, (_B, _S, _D)))
    perm = np.asarray(jax.random.permutation(k_perm, _B)).astype(np.int32)
    m = mask.reshape(_NSUB, 32, 16).astype(np.uint32)
    packed = (m << np.arange(32, dtype=np.uint32)[None, :, None]).sum(
        axis=1, dtype=np.uint32)
    pidx = (perm[:, None] * _SUB_PER_ROW
            + np.arange(_SUB_PER_ROW, dtype=np.int32)[None, :])
    return packed, pidx.reshape(_NSUB).astype(np.int32)


_PMASK, _PIDX = _build_tables()


@functools.cache
def _make_swap_kernel():
    mesh = plsc.VectorSubcoreMesh(core_axis_name="c", subcore_axis_name="s")
    nbuf = 2

    @functools.partial(
        pl.kernel,
        mesh=mesh,
        out_type=jax.ShapeDtypeStruct((_NSUB, _W), jnp.float32),
        scratch_types=[
            pltpu.VMEM((_SUB_PER_WORKER,), jnp.int32),
            *[pltpu.VMEM((_G, _W), jnp.float32) for _ in range(nbuf)],
            *[pltpu.VMEM((_G, _W), jnp.float32) for _ in range(nbuf)],
            *[pltpu.VMEM((_G, _W), jnp.float32) for _ in range(nbuf)],
            *[pltpu.VMEM((_G, 16), jnp.uint32) for _ in range(nbuf)],
            *[pltpu.SemaphoreType.DMA for _ in range(3 * nbuf)],
        ],
    )
    def _swap_kernel(x_hbm, pm_hbm, pidx_hbm, out_hbm, idx_v,
                     x_v0, x_v1, swap_v0, swap_v1, out_v0, out_v1,
                     p_v0, p_v1, in_s0, in_s1, g_s0, g_s1, o_s0, o_s1):
        x_vs, swap_vs, out_vs, p_vs = ((x_v0, x_v1), (swap_v0, swap_v1),
                                       (out_v0, out_v1), (p_v0, p_v1))
        in_sems, g_sems, o_sems = (in_s0, in_s1), (g_s0, g_s1), (o_s0, o_s1)
        wid = lax.axis_index("s") * _NC + lax.axis_index("c")
        base = wid * _SUB_PER_WORKER
        pltpu.sync_copy(pidx_hbm.at[pl.ds(base, _SUB_PER_WORKER)], idx_v)

        def issue_reads(b, gg):
            row0 = pl.multiple_of(base + gg * _G, _G)
            goff = pl.multiple_of(gg * _G, _G)
            pltpu.async_copy(x_hbm.at[pl.ds(row0, _G)], x_vs[b], in_sems[b])
            pltpu.async_copy(pm_hbm.at[pl.ds(row0, _G)], p_vs[b], in_sems[b])
            pltpu.async_copy(x_hbm.at[idx_v.at[pl.ds(goff, _G)]],
                             swap_vs[b], g_sems[b])

        def wait_reads(b):
            pltpu.make_async_copy(
                x_hbm.at[pl.ds(0, _G)], x_vs[b], in_sems[b]).wait()
            pltpu.make_async_copy(
                pm_hbm.at[pl.ds(0, _G)], p_vs[b], in_sems[b]).wait()
            pltpu.make_async_copy(
                x_hbm.at[idx_v.at[pl.ds(0, _G)]], swap_vs[b],
                g_sems[b]).wait()

        def wait_out(b):
            pltpu.make_async_copy(
                out_vs[b], out_hbm.at[pl.ds(0, _G)], o_sems[b]).wait()

        for b in range(nbuf):
            issue_reads(b, b)

        def pair(i, carry):
            for b in range(nbuf):
                gg = i * nbuf + b
                row0 = pl.multiple_of(base + gg * _G, _G)
                wait_reads(b)

                @pl.when(gg >= nbuf)
                def _():
                    wait_out(b)

                for r in range(_G):
                    pvec = p_vs[b][r, :]
                    for c in range(32):
                        off = c * 16
                        xc = x_vs[b][r, pl.ds(off, 16)]
                        sc = swap_vs[b][r, pl.ds(off, 16)]
                        m = (pvec & jnp.uint32(1 << c)) != 0
                        out_vs[b][r, pl.ds(off, 16)] = jnp.where(
                            m & (xc != 0.0), sc, xc)
                pltpu.async_copy(out_vs[b], out_hbm.at[pl.ds(row0, _G)],
                                 o_sems[b])

                @pl.when(gg + nbuf < _NGROUPS)
                def _():
                    issue_reads(b, gg + nbuf)
            return carry

        lax.fori_loop(0, _NGROUPS // nbuf, pair, 0)
        for b in range(nbuf):
            wait_out(b)

    return _swap_kernel


def kernel(inputs):
    x = inputs.reshape(_NSUB, _W)
    out = _make_swap_kernel()(x, jnp.asarray(_PMASK), jnp.asarray(_PIDX))
    return out.reshape(_B, _S, _D)


# speedup vs baseline: 1.7308x; 1.2398x over previous
import functools

import jax
import jax.numpy as jnp
import numpy as np
from jax import lax
from jax.experimental import pallas as pl
from jax.experimental.pallas import tpu as pltpu
from jax.experimental.pallas import tpu_sc as plsc

_B, _S, _D = 4096, 200, 64
_SPB = _S // 8
_NSLAB = _B * _SPB
_NC, _NS = 2, 16
_NW = _NC * _NS
_SLAB_PER_WORKER = _NSLAB // _NW
_G = 8
_NGROUPS = _SLAB_PER_WORKER // _G
_NGRP_TOT = _NSLAB // _G


def _tables():
    key = jax.random.key(42)
    k_mask, k_perm = jax.random.split(key)
    mask = np.asarray(jax.random.bernoulli(k_mask, 0.1, (_B, _S, _D)))
    perm = np.asarray(jax.random.permutation(k_perm, _B)).astype(np.int32)
    m = mask.reshape(_NSLAB, 32, 16).astype(np.uint32)
    packed = (m << np.arange(32, dtype=np.uint32)[None, :, None]).sum(
        axis=1, dtype=np.uint32)
    pm = packed.reshape(_NGRP_TOT, 1, 128)
    sidx = (perm[:, None] * _SPB
            + np.arange(_SPB, dtype=np.int32)[None, :])
    return pm, sidx.reshape(_NSLAB).astype(np.int32)


_PMASK, _SIDX = _tables()


@functools.cache
def _make_swap_kernel():
    mesh = plsc.VectorSubcoreMesh(core_axis_name="c", subcore_axis_name="s")
    nbuf = 2

    @functools.partial(
        pl.kernel,
        mesh=mesh,
        out_type=jax.ShapeDtypeStruct((_NSLAB, 8, _D), jnp.float32),
        scratch_types=[
            pltpu.VMEM((_SLAB_PER_WORKER,), jnp.int32),
            *[pltpu.VMEM((_G, 8, _D), jnp.float32) for _ in range(nbuf)],
            *[pltpu.VMEM((_G, 512), jnp.float32) for _ in range(nbuf)],
            *[pltpu.VMEM((_G, 8, _D), jnp.float32) for _ in range(nbuf)],
            *[pltpu.VMEM((1, 128), jnp.uint32) for _ in range(nbuf)],
            *[pltpu.SemaphoreType.DMA for _ in range(3 * nbuf)],
        ],
    )
    def _swap_kernel(x_hbm, x512_hbm, pm_hbm, sidx_hbm, out_hbm, idx_v,
                     x_v0, x_v1, swap_v0, swap_v1, out_v0, out_v1,
                     p_v0, p_v1, in_s0, in_s1, g_s0, g_s1, o_s0, o_s1):
        x_vs, swap_vs, out_vs, p_vs = ((x_v0, x_v1), (swap_v0, swap_v1),
                                       (out_v0, out_v1), (p_v0, p_v1))
        in_sems, g_sems, o_sems = (in_s0, in_s1), (g_s0, g_s1), (o_s0, o_s1)
        wid = lax.axis_index("s") * _NC + lax.axis_index("c")
        sbase = wid * _SLAB_PER_WORKER
        gbase = wid * _NGROUPS
        pltpu.sync_copy(sidx_hbm.at[pl.ds(sbase, _SLAB_PER_WORKER)], idx_v)

        def issue_reads(b, gg):
            slab0 = pl.multiple_of(sbase + gg * _G, _G)
            goff = pl.multiple_of(gg * _G, _G)
            pltpu.async_copy(x_hbm.at[pl.ds(slab0, _G)], x_vs[b], in_sems[b])
            pltpu.async_copy(pm_hbm.at[gbase + gg], p_vs[b], in_sems[b])
            pltpu.async_copy(x512_hbm.at[idx_v.at[pl.ds(goff, _G)]],
                             swap_vs[b], g_sems[b])

        def wait_reads(b):
            pltpu.make_async_copy(
                x_hbm.at[pl.ds(0, _G)], x_vs[b], in_sems[b]).wait()
            pltpu.make_async_copy(pm_hbm.at[0], p_vs[b], in_sems[b]).wait()
            pltpu.make_async_copy(
                x512_hbm.at[idx_v.at[pl.ds(0, _G)]], swap_vs[b],
                g_sems[b]).wait()

        def wait_out(b):
            pltpu.make_async_copy(
                out_vs[b], out_hbm.at[pl.ds(0, _G)], o_sems[b]).wait()

        for b in range(nbuf):
            issue_reads(b, b)

        def pair(i, carry):
            for b in range(nbuf):
                gg = i * nbuf + b
                slab0 = pl.multiple_of(sbase + gg * _G, _G)
                wait_reads(b)

                @pl.when(gg >= nbuf)
                def _():
                    wait_out(b)

                for r in range(_G):
                    pvec = p_vs[b][0, pl.ds(16 * r, 16)]
                    for c in range(32):
                        sr = c // 4
                        off = (c % 4) * 16
                        xc = x_vs[b][r, sr, pl.ds(off, 16)]
                        sc = swap_vs[b][r, pl.ds(c * 16, 16)]
                        m = (pvec & jnp.uint32(1 << c)) != 0
                        out_vs[b][r, sr, pl.ds(off, 16)] = jnp.where(
                            m & (xc != 0.0), sc, xc)
                pltpu.async_copy(out_vs[b], out_hbm.at[pl.ds(slab0, _G)],
                                 o_sems[b])

                @pl.when(gg + nbuf < _NGROUPS)
                def _():
                    issue_reads(b, gg + nbuf)
            return carry

        lax.fori_loop(0, _NGROUPS // nbuf, pair, 0)
        for b in range(nbuf):
            wait_out(b)

    return _swap_kernel


def kernel(inputs):
    x = inputs.reshape(_NSLAB, 8, _D)
    x512 = inputs.reshape(_NSLAB, 512)
    out = _make_swap_kernel()(x, x512, jnp.asarray(_PMASK),
                              jnp.asarray(_SIDX))
    return out.reshape(_B, _S, _D)
